# baseline (speedup 1.0000x reference)
import jax
import jax.numpy as jnp
from jax import lax
from jax.experimental import pallas as pl
from jax.experimental.pallas import tpu as pltpu
from jax.experimental.pallas import tpu_sc as plsc

_NC = 2
_NS = 16
_NW = _NC * _NS
_D = 64
_CHUNK = 200
_HALF = 100


def _build(nrows, nchunk):
    rpw = nrows // _NW
    mesh = plsc.VectorSubcoreMesh(
        core_axis_name="c", subcore_axis_name="s",
        num_cores=_NC, num_subcores=_NS)

    def body(x_hbm, tok_hbm, pos_hbm, out_hbm,
             idx_v, pos_v, g0, g1, o0, o1, gs0, gs1, os0, os1):
        w = lax.axis_index("s") * _NC + lax.axis_index("c")
        base = w * rpw
        pltpu.sync_copy(x_hbm.at[w], idx_v)
        pltpu.sync_copy(pos_hbm, pos_v)

        gbuf = (g0, g1)
        obuf = (o0, o1)
        gsem = (gs0, gs1)
        osem = (os0, os1)

        def start_gather(c, b):
            pltpu.async_copy(tok_hbm.at[idx_v.at[2 * c]],
                             gbuf[b].at[pl.ds(0, _HALF)], gsem[b])
            pltpu.async_copy(tok_hbm.at[idx_v.at[2 * c + 1]],
                             gbuf[b].at[pl.ds(_HALF, _HALF)], gsem[b])

        def wait_gather(c, b):
            pltpu.make_async_copy(tok_hbm.at[idx_v.at[2 * c]],
                                  gbuf[b].at[pl.ds(0, _HALF)], gsem[b]).wait()
            pltpu.make_async_copy(tok_hbm.at[idx_v.at[2 * c + 1]],
                                  gbuf[b].at[pl.ds(_HALF, _HALF)], gsem[b]).wait()

        def start_out(c, b):
            pltpu.async_copy(obuf[b],
                             out_hbm.at[pl.ds(base + c * _CHUNK, _CHUNK)],
                             osem[b])

        def wait_out(b):
            pltpu.make_async_copy(obuf[b],
                                  out_hbm.at[pl.ds(base, _CHUNK)],
                                  osem[b]).wait()

        def compute(b):
            src = gbuf[b]
            dst = obuf[b]

            def row(j, carry):
                for k in range(_D // 16):
                    sl = pl.ds(k * 16, 16)
                    dst[j, sl] = src[j, sl] + pos_v[j, sl]
                return carry

            lax.fori_loop(0, _CHUNK, row, 0, unroll=2)

        start_gather(0, 0)
        start_gather(1, 1)

        def step(g, carry):
            for h in range(2):
                c = 2 * g + h
                wait_gather(c, h)

                @pl.when(g >= 1)
                def _():
                    wait_out(h)

                compute(h)

                @pl.when(g < nchunk // 2 - 1)
                def _():
                    start_gather(c + 2, h)

                start_out(c, h)
            return carry

        lax.fori_loop(0, nchunk // 2, step, 0)
        wait_out(0)
        wait_out(1)

    return pl.kernel(
        body,
        out_type=jax.ShapeDtypeStruct((nrows, _D), jnp.float32),
        mesh=mesh,
        compiler_params=pltpu.CompilerParams(use_tc_tiling_on_sc=False),
        scratch_types=[
            pltpu.VMEM((2 * nchunk, _HALF), jnp.int32),
            pltpu.VMEM((_CHUNK, _D), jnp.float32),
            pltpu.VMEM((_CHUNK, _D), jnp.float32),
            pltpu.VMEM((_CHUNK, _D), jnp.float32),
            pltpu.VMEM((_CHUNK, _D), jnp.float32),
            pltpu.VMEM((_CHUNK, _D), jnp.float32),
            pltpu.SemaphoreType.DMA,
            pltpu.SemaphoreType.DMA,
            pltpu.SemaphoreType.DMA,
            pltpu.SemaphoreType.DMA,
        ],
    )


def kernel(x, token_table, pos_table):
    B, T = x.shape
    _, D = token_table.shape
    nrows = B * T
    nchunk = (nrows // _NW) // _CHUNK
    xr = x.reshape(_NW, 2 * nchunk, _HALF).astype(jnp.int32)
    out = _build(nrows, nchunk)(xr, token_table, pos_table)
    return out.reshape(B, T, D)
